# baseline (speedup 1.0000x reference)
import jax
import jax.numpy as jnp
from jax import lax
from jax.experimental import pallas as pl
from jax.experimental.pallas import tpu as pltpu
from jax.experimental.pallas import tpu_sc as plsc

B = 16384
S_DIM = 32
T_DIM = 32
R_DIM = 64
L = 16
NC = 2
NS = 16
NW = NC * NS
PER_W = B // NW
C = 64
NCHUNK = PER_W // C
NG = C // L
NITER = NG * T_DIM

_S1 = -1.6666654611e-1
_S2 = 8.3321608736e-3
_S3 = -1.9515295891e-4
_PI = 3.14159265358979323846
_INV_PI = 1.0 / _PI
_MAGIC = 12582912.0


def _sin_poly(r):
    r2 = r * r
    return r + r * r2 * (_S1 + r2 * (_S2 + r2 * _S3))


def _sin_reduced(x):
    n_f = (x * _INV_PI + _MAGIC) - _MAGIC
    r = x - n_f * _PI
    s = _sin_poly(r)
    odd = (n_f.astype(jnp.int32) & 1) == 1
    return jnp.where(odd, -s, s)


def _sc_body(*refs):
    samples = refs[0]
    tables = refs[1:3] + refs[5:23]
    rel_f = refs[3]
    rel_i = refs[4]
    out_hbm = refs[23]
    samp_v = refs[24]
    hidx = refs[25]
    ridx = refs[26]
    tidx = refs[27]
    out_v = refs[28]
    bufs = refs[29:69]
    rf_v = refs[69]
    ri_v = refs[70]
    sem = refs[71]


    wid = lax.axis_index("s") * NC + lax.axis_index("c")
    lane = lax.iota(jnp.int32, L)

    def chunk_body(c, carry):
        base = wid * PER_W + c * C
        pltpu.sync_copy(samples.at[pl.ds(base, C)], samp_v)

        def idx_body(g, carry2):
            ids = g * L + lane
            for col, dst in ((0, hidx), (1, ridx), (2, tidx)):
                cv = jnp.full((L,), col, jnp.int32)
                v = plsc.load_gather(samp_v, [ids, cv]).astype(jnp.int32)
                dst[pl.ds(g * L, L)] = v
            return carry2
        lax.fori_loop(0, NG, idx_body, 0)

        handles = []
        for k, tbl in enumerate(tables):
            handles.append(pltpu.async_copy(tbl.at[hidx], bufs[2 * k], sem))
            handles.append(pltpu.async_copy(tbl.at[tidx], bufs[2 * k + 1], sem))
        handles.append(pltpu.async_copy(rel_f.at[ridx], rf_v, sem))
        handles.append(pltpu.async_copy(rel_i.at[ridx], ri_v, sem))
        for h in handles:
            h.wait()

        for g in range(NG):
            out_v[pl.ds(g * L, L)] = jnp.zeros((L,), jnp.float32)

        def iter_body(k, carry2):
            g = lax.shift_right_logical(k, 5)
            d = lax.bitwise_and(k, T_DIM - 1)
            ids = g * L + lane
            dv = jnp.full((L,), d, jnp.int32)

            def col(buf, cvec):
                return plsc.load_gather(buf, [ids, cvec])

            year = col(samp_v, jnp.full((L,), 3, jnp.int32))
            month = col(samp_v, jnp.full((L,), 4, jnp.int32))
            day = col(samp_v, jnp.full((L,), 5, jnp.int32))

            def te(s, side):
                def tb(pos):
                    return col(bufs[2 * (pos + s) + side], dv)
                e = tb(18) * _sin_reduced(tb(6) * year + tb(12))
                e = e + tb(14) * _sin_poly(tb(2) * month + tb(8))
                e = e + tb(16) * _sin_poly(tb(4) * day + tb(10))
                return e

            r1s = col(rf_v, dv)
            r1t = col(rf_v, dv + S_DIM)
            r2s = col(ri_v, dv)
            r2t = col(ri_v, dv + S_DIM)

            eh_h = col(bufs[0], dv)
            eh_t = col(bufs[1], dv)
            et_h = col(bufs[2], dv)
            et_t = col(bufs[3], dv)

            contrib = eh_h * r1s * et_t + eh_t * r2s * et_h
            contrib = contrib + te(0, 0) * r1t * te(1, 1)
            contrib = contrib + te(0, 1) * r2t * te(1, 0)
            plsc.addupdate(out_v.at[pl.ds(g * L, L)], contrib)
            return carry2
        lax.fori_loop(0, NITER, iter_body, 0)

        for g in range(NG):
            sl = pl.ds(g * L, L)
            out_v[sl] = out_v[sl] * 0.5
        pltpu.sync_copy(out_v, out_hbm.at[pl.ds(base, C)])
        return carry

    lax.fori_loop(0, NCHUNK, chunk_body, 0)


def kernel(samples, ent_embs_h, ent_embs_t, rel_embs_f, rel_embs_i,
           m_freq_h, m_freq_t, d_freq_h, d_freq_t, y_freq_h, y_freq_t,
           m_phi_h, m_phi_t, d_phi_h, d_phi_t, y_phi_h, y_phi_t,
           m_amps_h, m_amps_t, d_amps_h, d_amps_t, y_amps_h, y_amps_t):
    mesh = plsc.VectorSubcoreMesh(core_axis_name="c", subcore_axis_name="s")
    scratch = (
        [pltpu.VMEM((C, 6), jnp.float32)]
        + [pltpu.VMEM((C,), jnp.int32)] * 3
        + [pltpu.VMEM((C,), jnp.float32)]
        + [pltpu.VMEM((C, S_DIM), jnp.float32)] * 40
        + [pltpu.VMEM((C, R_DIM), jnp.float32)] * 2
        + [pltpu.SemaphoreType.DMA]
    )
    run = pl.kernel(
        _sc_body,
        mesh=mesh,
        out_type=jax.ShapeDtypeStruct((B,), jnp.float32),
        scratch_types=scratch,
        compiler_params=pltpu.CompilerParams(
            needs_layout_passes=False, use_tc_tiling_on_sc=False),
    )
    return run(samples, ent_embs_h, ent_embs_t, rel_embs_f, rel_embs_i,
               m_freq_h, m_freq_t, d_freq_h, d_freq_t, y_freq_h, y_freq_t,
               m_phi_h, m_phi_t, d_phi_h, d_phi_t, y_phi_h, y_phi_t,
               m_amps_h, m_amps_t, d_amps_h, d_amps_t, y_amps_h, y_amps_t)


# speedup vs baseline: 1.0912x; 1.0912x over previous
import jax
import jax.numpy as jnp
from jax import lax
from jax.experimental import pallas as pl
from jax.experimental.pallas import tpu as pltpu
from jax.experimental.pallas import tpu_sc as plsc

B = 16384
NUM_ENT = 100000
NUM_REL = 1000
S_DIM = 32
R_DIM = 64
GW = 128
L = 16
NC = 2
NS = 16
NW = NC * NS

RCHUNK = 40
NCHUNK1 = NUM_ENT // RCHUNK
ITER1 = (NCHUNK1 + NW - 1) // NW
RELCHUNK = NUM_REL // 25

C = 64
PER_W = B // NW
NCHUNK2 = PER_W // C

_S1 = -1.6666654611e-1
_S2 = 8.3321608736e-3
_S3 = -1.9515295891e-4
_PI = 3.14159265358979323846
_INV_PI = 1.0 / _PI
_MAGIC = 12582912.0


def _sin_poly(r):
    r2 = r * r
    return r + r * r2 * (_S1 + r2 * (_S2 + r2 * _S3))


def _sin_reduced(x):
    n_f = (x * _INV_PI + _MAGIC) - _MAGIC
    r = x - n_f * _PI
    s = _sin_poly(r)
    odd = (n_f.astype(jnp.int32) & 1) == 1
    return jnp.where(odd, -s, s)


def _repack4_body(*refs):
    tables = refs[0:4]
    packed = refs[4]
    rbufs = refs[5:9]
    stage = refs[9]
    sem = refs[10]

    wid = lax.axis_index("s") * NC + lax.axis_index("c")

    def chunk_body(j, carry):
        c = j * NW + wid

        @pl.when(c < NCHUNK1)
        def _():
            base = c * RCHUNK
            handles = [
                pltpu.async_copy(tbl.at[pl.ds(base, RCHUNK)], rbufs[t], sem)
                for t, tbl in enumerate(tables)
            ]
            for h in handles:
                h.wait()

            def row_body(r, carry2):
                for t in range(4):
                    for q in range(2):
                        stage[r, pl.ds(S_DIM * t + L * q, L)] = (
                            rbufs[t][r, pl.ds(L * q, L)])
                return carry2
            lax.fori_loop(0, RCHUNK, row_body, 0)
            pltpu.sync_copy(stage, packed.at[pl.ds(base, RCHUNK)])
        return carry

    lax.fori_loop(0, ITER1, chunk_body, 0)


def _repack4_rel_body(*refs):
    rel_f = refs[4]
    rel_i = refs[5]
    relp = refs[7]
    rfb = refs[13]
    rib = refs[14]
    rstage = refs[15]
    sem = refs[16]

    _repack4_body(*(refs[0:4] + refs[6:7] + refs[8:13] + refs[16:17]))

    wid = lax.axis_index("s") * NC + lax.axis_index("c")

    @pl.when(wid < 25)
    def _():
        base = wid * RELCHUNK
        h1 = pltpu.async_copy(rel_f.at[pl.ds(base, RELCHUNK)], rfb, sem)
        h2 = pltpu.async_copy(rel_i.at[pl.ds(base, RELCHUNK)], rib, sem)
        h1.wait()
        h2.wait()

        def rel_row(r, carry2):
            for q in range(4):
                rstage[r, pl.ds(L * q, L)] = rfb[r, pl.ds(L * q, L)]
                rstage[r, pl.ds(R_DIM + L * q, L)] = rib[r, pl.ds(L * q, L)]
            return carry2
        lax.fori_loop(0, RELCHUNK, rel_row, 0)
        pltpu.sync_copy(rstage, relp.at[pl.ds(base, RELCHUNK)])


def _score_body(*refs):
    samples = refs[0]
    packed = refs[1:6]
    relp = refs[6]
    out_hbm = refs[7]
    samp_v = refs[8]
    hidx = refs[9]
    ridx = refs[10]
    tidx = refs[11]
    out_v = refs[12]
    hbufs = refs[13:18]
    tbufs = refs[18:23]
    rbuf = refs[23]
    sem = refs[24]

    wid = lax.axis_index("s") * NC + lax.axis_index("c")
    lane = lax.iota(jnp.int32, L)
    lane0 = lane == 0

    def chunk_body(cc, carry):
        base = wid * PER_W + cc * C
        pltpu.sync_copy(samples.at[pl.ds(base, C)], samp_v)

        def idx_body(g, carry2):
            ids = g * L + lane
            for col, dst in ((0, hidx), (1, ridx), (2, tidx)):
                cv = jnp.full((L,), col, jnp.int32)
                v = plsc.load_gather(samp_v, [ids, cv]).astype(jnp.int32)
                dst[pl.ds(g * L, L)] = v
            return carry2
        lax.fori_loop(0, C // L, idx_body, 0)

        handles = []
        for g in range(5):
            handles.append(pltpu.async_copy(packed[g].at[hidx], hbufs[g], sem))
            handles.append(pltpu.async_copy(packed[g].at[tidx], tbufs[g], sem))
        handles.append(pltpu.async_copy(relp.at[ridx], rbuf, sem))
        for h in handles:
            h.wait()

        def sample_body(i, carry2):
            iv = jnp.full((L,), i, jnp.int32)
            year = plsc.load_gather(samp_v, [iv, jnp.full((L,), 3, jnp.int32)])
            month = plsc.load_gather(samp_v, [iv, jnp.full((L,), 4, jnp.int32)])
            day = plsc.load_gather(samp_v, [iv, jnp.full((L,), 5, jnp.int32)])

            acc = jnp.zeros((L,), jnp.float32)
            for q in range(2):
                qo = L * q

                def tcol(bufs, t):
                    return bufs[t // 4][i, pl.ds(S_DIM * (t % 4) + qo, L)]

                def te(s, bufs):
                    e = tcol(bufs, 18 + s) * _sin_reduced(
                        tcol(bufs, 6 + s) * year + tcol(bufs, 12 + s))
                    e = e + tcol(bufs, 14 + s) * _sin_poly(
                        tcol(bufs, 2 + s) * month + tcol(bufs, 8 + s))
                    e = e + tcol(bufs, 16 + s) * _sin_poly(
                        tcol(bufs, 4 + s) * day + tcol(bufs, 10 + s))
                    return e

                r1s = rbuf[i, pl.ds(qo, L)]
                r1t = rbuf[i, pl.ds(S_DIM + qo, L)]
                r2s = rbuf[i, pl.ds(R_DIM + qo, L)]
                r2t = rbuf[i, pl.ds(R_DIM + S_DIM + qo, L)]

                acc = acc + tcol(hbufs, 0) * r1s * tcol(tbufs, 1)
                acc = acc + tcol(tbufs, 0) * r2s * tcol(hbufs, 1)
                acc = acc + te(0, hbufs) * r1t * te(1, tbufs)
                acc = acc + te(0, tbufs) * r2t * te(1, hbufs)

            score = jnp.sum(acc) * 0.5
            plsc.store_scatter(out_v, [iv], jnp.full((L,), score), mask=lane0)
            return carry2
        lax.fori_loop(0, C, sample_body, 0)

        pltpu.sync_copy(out_v, out_hbm.at[pl.ds(base, C)])
        return carry

    lax.fori_loop(0, NCHUNK2, chunk_body, 0)


def kernel(samples, ent_embs_h, ent_embs_t, rel_embs_f, rel_embs_i,
           m_freq_h, m_freq_t, d_freq_h, d_freq_t, y_freq_h, y_freq_t,
           m_phi_h, m_phi_t, d_phi_h, d_phi_t, y_phi_h, y_phi_t,
           m_amps_h, m_amps_t, d_amps_h, d_amps_t, y_amps_h, y_amps_t):
    mesh = plsc.VectorSubcoreMesh(core_axis_name="c", subcore_axis_name="s")
    params = pltpu.CompilerParams(needs_layout_passes=False)
    fpk = jax.ShapeDtypeStruct((NUM_ENT, GW), jnp.float32)

    groups = [
        (ent_embs_h, ent_embs_t, m_freq_h, m_freq_t),
        (d_freq_h, d_freq_t, y_freq_h, y_freq_t),
        (m_phi_h, m_phi_t, d_phi_h, d_phi_t),
        (y_phi_h, y_phi_t, m_amps_h, m_amps_t),
        (d_amps_h, d_amps_t, y_amps_h, y_amps_t),
    ]

    base_scratch = (
        [pltpu.VMEM((RCHUNK, S_DIM), jnp.float32)] * 4
        + [pltpu.VMEM((RCHUNK, GW), jnp.float32)]
    )
    repack4 = pl.kernel(
        _repack4_body,
        mesh=mesh,
        out_type=fpk,
        scratch_types=base_scratch + [pltpu.SemaphoreType.DMA],
        compiler_params=params,
    )
    repack4_rel = pl.kernel(
        _repack4_rel_body,
        mesh=mesh,
        out_type=(fpk, jax.ShapeDtypeStruct((NUM_REL, 2 * R_DIM), jnp.float32)),
        scratch_types=(
            base_scratch
            + [pltpu.VMEM((RELCHUNK, R_DIM), jnp.float32)] * 2
            + [pltpu.VMEM((RELCHUNK, 2 * R_DIM), jnp.float32)]
            + [pltpu.SemaphoreType.DMA]
        ),
        compiler_params=params,
    )

    packed = [repack4(*groups[g]) for g in range(4)]
    pk4, relp = repack4_rel(*groups[4], rel_embs_f, rel_embs_i)
    packed.append(pk4)

    score = pl.kernel(
        _score_body,
        mesh=mesh,
        out_type=jax.ShapeDtypeStruct((B,), jnp.float32),
        scratch_types=(
            [pltpu.VMEM((C, 6), jnp.float32)]
            + [pltpu.VMEM((C,), jnp.int32)] * 3
            + [pltpu.VMEM((C,), jnp.float32)]
            + [pltpu.VMEM((C, GW), jnp.float32)] * 10
            + [pltpu.VMEM((C, 2 * R_DIM), jnp.float32)]
            + [pltpu.SemaphoreType.DMA]
        ),
        compiler_params=params,
    )
    return score(samples, *packed, relp)
